# baseline (speedup 1.0000x reference)
import functools

import jax
import jax.numpy as jnp
from jax import lax
from jax.experimental import pallas as pl
from jax.experimental.pallas import tpu as pltpu
from jax.experimental.pallas import tpu_sc as plsc

N_NODES = 50000
N_EDGES = 800000
NODE_VOCAB = 100000
EDGE_VOCAB = 1000
D = 128
DE = 16
BN_EPS = 1e-5

NC = 2
NS = 16
NW = NC * NS
L = 16

N_NHI = 10
NROW_HI = 1568
NROW_LO = 1560
NAP_CH = 192
NAP_NC = 9
NAP_FULL = NAP_CH * (NAP_NC - 1)
NAP_TAIL = 32

EPW = N_EDGES // NW

ETILES = N_EDGES // 128
ET_HI = 196
ET_LO = 195
ET_NHI = ETILES - NW * ET_LO
ECHT = 8
EFC = 24
EAP_NC = 25
ET_FULL = EFC * ECHT
ET_TAIL = 4


def _worker_bases(wid):
    nbase = wid * NROW_LO + jnp.minimum(wid, N_NHI) * 8
    ntail = jnp.where(wid < N_NHI, NAP_FULL, NAP_FULL - 8)
    etb0 = wid * ET_LO + jnp.minimum(wid, ET_NHI)
    etail = jnp.where(wid < ET_NHI, ET_FULL, ET_FULL - 1)
    return nbase, ntail, etb0, etail

EVP = 1024
HWORDS = EVP * L

_mesh = plsc.VectorSubcoreMesh(
    core_axis_name="c", subcore_axis_name="s", num_cores=NC, num_subcores=NS)


def _wid():
    return lax.axis_index("s") * NC + lax.axis_index("c")


def _stats_body(nf_hbm, ea_hbm, tbl_hbm, nsum_out, nsq_out, ecnt_out,
                nidx_v, rows_a, rows_b, eidx_v, h_v, cnt_v, acc_v,
                sem_ni, sem_ei, sem_g0, sem_g1, sem_out):
    wid = _wid()
    lanes = lax.iota(jnp.int32, L)
    zf = jnp.zeros((L,), jnp.float32)
    zi = jnp.zeros((L,), jnp.int32)
    onesi = jnp.ones((L,), jnp.int32)
    rows2 = (rows_a, rows_b)
    sem_g = (sem_g0, sem_g1)

    nbase, ntail, _, _ = _worker_bases(wid)

    eidx_v[pl.ds((EPW // L) * L, L)] = zi
    ebase = wid * EPW
    ei_d = pltpu.async_copy(
        ea_hbm.at[pl.ds(ebase, EPW)], eidx_v.at[pl.ds(0, EPW)], sem_ei)
    ni_d = pltpu.async_copy(
        nf_hbm.at[pl.ds(nbase, NAP_FULL)], nidx_v.at[pl.ds(0, NAP_FULL)],
        sem_ni)
    nt_d = pltpu.async_copy(
        nf_hbm.at[pl.ds(nbase + ntail, NAP_TAIL)],
        nidx_v.at[pl.ds(NAP_FULL, NAP_TAIL)], sem_ni)

    def zero_h(i, _):
        h_v[pl.ds(i * L, L)] = zi
        return 0
    lax.fori_loop(0, HWORDS // L, zero_h, 0)

    gd = {}

    def n_gather_start(p):
        sz = NAP_CH if p < NAP_NC - 1 else NAP_TAIL
        off = p * NAP_CH
        descs = []
        for j in range(0, sz, 128):
            w = min(128, sz - j)
            descs.append(pltpu.async_copy(
                tbl_hbm.at[nidx_v.at[pl.ds(off + j, w)]],
                rows2[p % 2].at[pl.ds(j, w)], sem_g[p % 2]))
        return descs

    ni_d.wait()
    nt_d.wait()
    gd[0] = n_gather_start(0)
    gd[1] = n_gather_start(1)

    ei_d.wait()

    def hist(k, _):
        v = eidx_v[pl.ds(k * L, L)]
        plsc.addupdate_scatter(h_v, [v * L + lanes], onesi, mask=lanes >= 0)
        return 0
    lax.fori_loop(0, EPW // L, hist, 0)
    netail = EPW - (EPW // L) * L
    vtail = eidx_v[pl.ds((EPW // L) * L, L)]
    plsc.addupdate_scatter(h_v, [vtail * L + lanes], onesi,
                           mask=lanes < netail)

    def red(b, _):
        acc = zi
        base = (b * L + lanes) * L
        for l in range(L):
            acc = acc + plsc.load_gather(h_v, [base + l])
        cnt_v[pl.ds(b * L, L)] = acc
        return 0
    lax.fori_loop(0, EVP // L, red, 0)
    cnt_d = pltpu.async_copy(cnt_v, ecnt_out.at[wid], sem_out)

    car = tuple(zf for _ in range(2 * (D // L)))
    for p in range(NAP_NC):
        for dsc in gd[p]:
            dsc.wait()
        buf = rows2[p % 2]

        def acc_row(r, c_, buf=buf):
            new = []
            for c in range(D // L):
                v = buf[r, pl.ds(c * L, L)]
                new.append(c_[c] + v)
            for c in range(D // L):
                v = buf[r, pl.ds(c * L, L)]
                new.append(c_[D // L + c] + v * v)
            return tuple(new)

        if p < NAP_NC - 1:
            car = lax.fori_loop(0, NAP_CH, acc_row, car)
        else:
            s0 = NAP_FULL - ntail
            car = lax.fori_loop(s0, NAP_TAIL, acc_row, car)
        if p + 2 < NAP_NC:
            gd[p + 2] = n_gather_start(p + 2)

    for c in range(D // L):
        acc_v[pl.ds(c * L, L)] = car[c]
        acc_v[pl.ds(D + c * L, L)] = car[D // L + c]
    pltpu.async_copy(acc_v.at[pl.ds(0, D)], nsum_out.at[wid], sem_out).wait()
    pltpu.async_copy(acc_v.at[pl.ds(D, D)], nsq_out.at[wid], sem_out).wait()
    cnt_d.wait()


_stats_kernel = functools.partial(
    pl.kernel,
    compiler_params=pltpu.CompilerParams(needs_layout_passes=False),
    out_type=(
        jax.ShapeDtypeStruct((NW, D), jnp.float32),
        jax.ShapeDtypeStruct((NW, D), jnp.float32),
        jax.ShapeDtypeStruct((NW, EVP), jnp.int32),
    ),
    mesh=_mesh,
    scratch_types=[
        pltpu.VMEM((NROW_HI,), jnp.int32),
        pltpu.VMEM((NAP_CH, D), jnp.float32),
        pltpu.VMEM((NAP_CH, D), jnp.float32),
        pltpu.VMEM((EPW + L,), jnp.int32),
        pltpu.VMEM((HWORDS,), jnp.int32),
        pltpu.VMEM((EVP,), jnp.int32),
        pltpu.VMEM((2 * D,), jnp.float32),
    ] + [pltpu.SemaphoreType.DMA] * 5,
)(_stats_body)


def _apply_body(nf_hbm, ea_hbm, tbl_hbm, coef_hbm, ntt_hbm, x_out, et_out,
                nidx_v, rows_a, rows_b, rows_c, eidx_a, eidx_b, etf_a, etf_b,
                ntt_v, coef_v, sem_ni, sem_g0, sem_g1, sem_g2, sem_o0, sem_o1,
                sem_o2, sem_ei0, sem_ei1, sem_eo0, sem_eo1):
    wid = _wid()
    pltpu.sync_copy(coef_hbm, coef_v)
    scales = [coef_v[pl.ds(c * L, L)] for c in range(D // L)]
    shifts = [coef_v[pl.ds(D + c * L, L)] for c in range(D // L)]
    pltpu.sync_copy(ntt_hbm, ntt_v)

    rows3 = (rows_a, rows_b, rows_c)
    eidx2 = (eidx_a, eidx_b)
    etf2 = (etf_a, etf_b)
    sem_g = (sem_g0, sem_g1, sem_g2)
    sem_o = (sem_o0, sem_o1, sem_o2)
    sem_ei = (sem_ei0, sem_ei1)
    sem_eo = (sem_eo0, sem_eo1)

    nbase, ntail, etb0, etail = _worker_bases(wid)
    gd, nod, ed, od = {}, {}, {}, {}

    def e_loc(q):
        return q * ECHT if q < EAP_NC - 1 else etail

    def e_sz(q):
        return ECHT if q < EAP_NC - 1 else ET_TAIL

    def e_idx_start(q):
        n = e_sz(q) * 128
        return pltpu.async_copy(
            ea_hbm.at[pl.ds((etb0 + e_loc(q)) * 128, n)],
            eidx2[q % 2].at[pl.ds(0, n)], sem_ei[q % 2])

    def e_compute(q):
        buf = etf2[q % 2]
        eb = eidx2[q % 2]

        def grp2(i, _):
            for u in range(2):
                k = i * 2 + u
                attr = eb[pl.ds(k * L, L)]
                s = ((k >> 3) << 10) + ((k & 7) << 4)
                for d in range(DE):
                    col = plsc.load_gather(ntt_v, [attr + d * EDGE_VOCAB])
                    buf[pl.ds(s + (d // 8) * (ECHT * 1024) + (d % 8) * 128,
                              L)] = col
            return 0
        lax.fori_loop(0, e_sz(q) * 4, grp2, 0)

    def e_out_start(q):
        return [pltpu.async_copy(
            etf2[q % 2].at[pl.ds(tr * ECHT * 1024, e_sz(q) * 1024)],
            et_out.at[pl.ds(tr * (ETILES * 1024) + (etb0 + e_loc(q)) * 1024,
                            e_sz(q) * 1024)],
            sem_eo[q % 2]) for tr in range(2)]

    def n_loc(p):
        return p * NAP_CH if p < NAP_NC - 1 else ntail

    def n_sz(p):
        return NAP_CH if p < NAP_NC - 1 else NAP_TAIL

    def n_gather_start(p):
        sz = n_sz(p)
        off = p * NAP_CH
        descs = []
        for j in range(0, sz, 128):
            w = min(128, sz - j)
            descs.append(pltpu.async_copy(
                tbl_hbm.at[nidx_v.at[pl.ds(off + j, w)]],
                rows3[p % 3].at[pl.ds(j, w)], sem_g[p % 3]))
        return descs

    def n_normalize(p):
        buf = rows3[p % 3]

        def norm_row2(i, _):
            for u in range(2):
                r = i * 2 + u
                for c in range(D // L):
                    v = buf[r, pl.ds(c * L, L)]
                    buf[r, pl.ds(c * L, L)] = v * scales[c] + shifts[c]
            return 0
        lax.fori_loop(0, n_sz(p) // 2, norm_row2, 0)

    def n_out_start(p):
        return pltpu.async_copy(
            rows3[p % 3].at[pl.ds(0, n_sz(p))],
            x_out.at[pl.ds(nbase + n_loc(p), n_sz(p))], sem_o[p % 3])

    def edge_step(q):
        ed[q].wait()
        if q >= 2:
            for dsc in od[q - 2]:
                dsc.wait()
        e_compute(q)
        od[q] = e_out_start(q)
        if q + 2 < EAP_NC:
            ed[q + 2] = e_idx_start(q + 2)

    def node_step(p):
        for dsc in gd[p]:
            dsc.wait()
        if p + 2 < NAP_NC:
            if p >= 1:
                nod[p - 1].wait()
            gd[p + 2] = n_gather_start(p + 2)
        n_normalize(p)
        nod[p] = n_out_start(p)

    ni_d = pltpu.async_copy(
        nf_hbm.at[pl.ds(nbase, NAP_FULL)], nidx_v.at[pl.ds(0, NAP_FULL)],
        sem_ni)
    nt_d = pltpu.async_copy(
        nf_hbm.at[pl.ds(nbase + ntail, NAP_TAIL)],
        nidx_v.at[pl.ds(NAP_FULL, NAP_TAIL)], sem_ni)
    ed[0] = e_idx_start(0)
    ed[1] = e_idx_start(1)
    ni_d.wait()
    nt_d.wait()
    gd[0] = n_gather_start(0)
    gd[1] = n_gather_start(1)

    p = 0
    for q in range(EAP_NC):
        edge_step(q)
        if q % 3 == 1 and p < NAP_NC:
            node_step(p)
            p += 1
    while p < NAP_NC:
        node_step(p)
        p += 1

    for q in (EAP_NC - 2, EAP_NC - 1):
        for dsc in od[q]:
            dsc.wait()
    nod[NAP_NC - 2].wait()
    nod[NAP_NC - 1].wait()


_apply_kernel = functools.partial(
    pl.kernel,
    compiler_params=pltpu.CompilerParams(needs_layout_passes=False),
    out_type=(
        jax.ShapeDtypeStruct((N_NODES, D), jnp.float32),
        jax.ShapeDtypeStruct((2 * ETILES * 8 * 128,), jnp.float32),
    ),
    mesh=_mesh,
    scratch_types=[
        pltpu.VMEM((NROW_HI,), jnp.int32),
        pltpu.VMEM((NAP_CH, D), jnp.float32),
        pltpu.VMEM((NAP_CH, D), jnp.float32),
        pltpu.VMEM((NAP_CH, D), jnp.float32),
        pltpu.VMEM((ECHT * 128,), jnp.int32),
        pltpu.VMEM((ECHT * 128,), jnp.int32),
        pltpu.VMEM((2 * ECHT * 8 * 128,), jnp.float32),
        pltpu.VMEM((2 * ECHT * 8 * 128,), jnp.float32),
        pltpu.VMEM((DE * EDGE_VOCAB,), jnp.float32),
        pltpu.VMEM((2 * D,), jnp.float32),
    ] + [pltpu.SemaphoreType.DMA] * 11,
)(_apply_body)


def kernel(node_feat, edge_index, edge_attr, node_table, edge_table,
           node_gamma, node_beta, edge_gamma, edge_beta):
    del edge_index
    nsum, nsq, ecnt = _stats_kernel(node_feat, edge_attr, node_table)

    ns = nsum.sum(axis=0)
    nq = nsq.sum(axis=0)
    nmean = ns / N_NODES
    nvar = jnp.maximum(nq / N_NODES - nmean * nmean, 0.0)
    nscale = node_gamma / jnp.sqrt(nvar + BN_EPS)
    nshift = node_beta - nmean * nscale
    coef = jnp.concatenate([nscale, nshift])

    cnt = ecnt.sum(axis=0)[:EDGE_VOCAB].astype(jnp.float32)
    es = cnt @ edge_table
    eq = cnt @ (edge_table * edge_table)
    emean = es / N_EDGES
    evar = jnp.maximum(eq / N_EDGES - emean * emean, 0.0)
    escale = edge_gamma / jnp.sqrt(evar + BN_EPS)
    nt = edge_table * escale + (edge_beta - emean * escale)
    ntt = nt.T.reshape(-1)

    x, et = _apply_kernel(node_feat, edge_attr, node_table, coef, ntt)
    e = (et.reshape(2, ETILES, 8, 128)
         .transpose(1, 3, 0, 2).reshape(N_EDGES, DE))
    return (x, e)


# speedup vs baseline: 1.8917x; 1.8917x over previous
import functools

import jax
import jax.numpy as jnp
from jax import lax
from jax.experimental import pallas as pl
from jax.experimental.pallas import tpu as pltpu
from jax.experimental.pallas import tpu_sc as plsc

N_NODES = 50000
N_EDGES = 800000
NODE_VOCAB = 100000
EDGE_VOCAB = 1000
D = 128
DE = 16
BN_EPS = 1e-5

NC = 2
NS = 16
NW = NC * NS
L = 16

N_NHI = 10
NROW_HI = 1568
NROW_LO = 1560
NAP_CH = 192
NAP_NC = 9
NAP_FULL = NAP_CH * (NAP_NC - 1)
NAP_TAIL = 32

EPW = N_EDGES // NW

ETILES = N_EDGES // 128
ET_HI = 196
ET_LO = 195
ET_NHI = ETILES - NW * ET_LO
ECHT = 8
EFC = 24
EAP_NC = 25
ET_FULL = EFC * ECHT
ET_TAIL = 4


def _worker_bases(wid):
    nbase = wid * NROW_LO + jnp.minimum(wid, N_NHI) * 8
    ntail = jnp.where(wid < N_NHI, NAP_FULL, NAP_FULL - 8)
    etb0 = wid * ET_LO + jnp.minimum(wid, ET_NHI)
    etail = jnp.where(wid < ET_NHI, ET_FULL, ET_FULL - 1)
    return nbase, ntail, etb0, etail

EVP = 1024
HWORDS = EVP * L

_mesh = plsc.VectorSubcoreMesh(
    core_axis_name="c", subcore_axis_name="s", num_cores=NC, num_subcores=NS)


def _wid():
    return lax.axis_index("s") * NC + lax.axis_index("c")


def _stats_body(nf_hbm, ea_hbm, tbl_hbm, nsum_out, nsq_out, ecnt_out,
                nidx_v, rows_a, rows_b, eidx_v, h_v, cnt_v, acc_v,
                sem_ni, sem_ei, sem_g0, sem_g1, sem_out):
    wid = _wid()
    lanes = lax.iota(jnp.int32, L)
    zf = jnp.zeros((L,), jnp.float32)
    zi = jnp.zeros((L,), jnp.int32)
    onesi = jnp.ones((L,), jnp.int32)
    rows2 = (rows_a, rows_b)
    sem_g = (sem_g0, sem_g1)

    nbase, ntail, _, _ = _worker_bases(wid)

    eidx_v[pl.ds((EPW // L) * L, L)] = zi
    ebase = wid * EPW
    ei_d = pltpu.async_copy(
        ea_hbm.at[pl.ds(ebase, EPW)], eidx_v.at[pl.ds(0, EPW)], sem_ei)
    ni_d = pltpu.async_copy(
        nf_hbm.at[pl.ds(nbase, NAP_FULL)], nidx_v.at[pl.ds(0, NAP_FULL)],
        sem_ni)
    nt_d = pltpu.async_copy(
        nf_hbm.at[pl.ds(nbase + ntail, NAP_TAIL)],
        nidx_v.at[pl.ds(NAP_FULL, NAP_TAIL)], sem_ni)

    def zero_h(i, _):
        h_v[pl.ds(i * L, L)] = zi
        return 0
    lax.fori_loop(0, HWORDS // L, zero_h, 0)

    gd = {}

    def n_gather_start(p):
        sz = NAP_CH if p < NAP_NC - 1 else NAP_TAIL
        off = p * NAP_CH
        descs = []
        for j in range(0, sz, 128):
            w = min(128, sz - j)
            descs.append(pltpu.async_copy(
                tbl_hbm.at[nidx_v.at[pl.ds(off + j, w)]],
                rows2[p % 2].at[pl.ds(j, w)], sem_g[p % 2]))
        return descs

    ni_d.wait()
    nt_d.wait()
    gd[0] = n_gather_start(0)
    gd[1] = n_gather_start(1)

    ei_d.wait()

    def hist(k, _):
        v = eidx_v[pl.ds(k * L, L)]
        plsc.addupdate_scatter(h_v, [v * L + lanes], onesi, mask=lanes >= 0)
        return 0
    lax.fori_loop(0, EPW // L, hist, 0)
    netail = EPW - (EPW // L) * L
    vtail = eidx_v[pl.ds((EPW // L) * L, L)]
    plsc.addupdate_scatter(h_v, [vtail * L + lanes], onesi,
                           mask=lanes < netail)

    def red(b, _):
        acc = zi
        base = (b * L + lanes) * L
        for l in range(L):
            acc = acc + plsc.load_gather(h_v, [base + l])
        cnt_v[pl.ds(b * L, L)] = acc
        return 0
    lax.fori_loop(0, EVP // L, red, 0)
    cnt_d = pltpu.async_copy(cnt_v, ecnt_out.at[wid], sem_out)

    car = tuple(zf for _ in range(2 * (D // L)))
    for p in range(NAP_NC):
        for dsc in gd[p]:
            dsc.wait()
        buf = rows2[p % 2]

        def acc_row(r, c_, buf=buf):
            new = []
            for c in range(D // L):
                v = buf[r, pl.ds(c * L, L)]
                new.append(c_[c] + v)
            for c in range(D // L):
                v = buf[r, pl.ds(c * L, L)]
                new.append(c_[D // L + c] + v * v)
            return tuple(new)

        if p < NAP_NC - 1:
            car = lax.fori_loop(0, NAP_CH, acc_row, car)
        else:
            s0 = NAP_FULL - ntail
            car = lax.fori_loop(s0, NAP_TAIL, acc_row, car)
        if p + 2 < NAP_NC:
            gd[p + 2] = n_gather_start(p + 2)

    for c in range(D // L):
        acc_v[pl.ds(c * L, L)] = car[c]
        acc_v[pl.ds(D + c * L, L)] = car[D // L + c]
    pltpu.async_copy(acc_v.at[pl.ds(0, D)], nsum_out.at[wid], sem_out).wait()
    pltpu.async_copy(acc_v.at[pl.ds(D, D)], nsq_out.at[wid], sem_out).wait()
    cnt_d.wait()


_stats_kernel = functools.partial(
    pl.kernel,
    compiler_params=pltpu.CompilerParams(needs_layout_passes=False),
    out_type=(
        jax.ShapeDtypeStruct((NW, D), jnp.float32),
        jax.ShapeDtypeStruct((NW, D), jnp.float32),
        jax.ShapeDtypeStruct((NW, EVP), jnp.int32),
    ),
    mesh=_mesh,
    scratch_types=[
        pltpu.VMEM((NROW_HI,), jnp.int32),
        pltpu.VMEM((NAP_CH, D), jnp.float32),
        pltpu.VMEM((NAP_CH, D), jnp.float32),
        pltpu.VMEM((EPW + L,), jnp.int32),
        pltpu.VMEM((HWORDS,), jnp.int32),
        pltpu.VMEM((EVP,), jnp.int32),
        pltpu.VMEM((2 * D,), jnp.float32),
    ] + [pltpu.SemaphoreType.DMA] * 5,
)(_stats_body)


def _apply_body(nf_hbm, ea_hbm, tbl_hbm, coef_hbm, ntt_hbm, x_out, et_out,
                nidx_v, rows_a, rows_b, rows_c, eidx_a, eidx_b, etf_a, etf_b,
                ntt_v, coef_v, sem_ni, sem_g0, sem_g1, sem_g2, sem_o0, sem_o1,
                sem_o2, sem_ei0, sem_ei1, sem_eo0, sem_eo1):
    wid = _wid()
    pltpu.sync_copy(coef_hbm, coef_v)
    scales = [coef_v[pl.ds(c * L, L)] for c in range(D // L)]
    shifts = [coef_v[pl.ds(D + c * L, L)] for c in range(D // L)]
    pltpu.sync_copy(ntt_hbm, ntt_v)

    rows3 = (rows_a, rows_b, rows_c)
    eidx2 = (eidx_a, eidx_b)
    etf2 = (etf_a, etf_b)
    sem_g = (sem_g0, sem_g1, sem_g2)
    sem_o = (sem_o0, sem_o1, sem_o2)
    sem_ei = (sem_ei0, sem_ei1)
    sem_eo = (sem_eo0, sem_eo1)

    nbase, ntail, etb0, etail = _worker_bases(wid)
    gd, nod, ed, od = {}, {}, {}, {}

    def e_loc(q):
        return q * ECHT if q < EAP_NC - 1 else etail

    def e_sz(q):
        return ECHT if q < EAP_NC - 1 else ET_TAIL

    def e_idx_start(q):
        n = e_sz(q) * 128
        return pltpu.async_copy(
            ea_hbm.at[pl.ds((etb0 + e_loc(q)) * 128, n)],
            eidx2[q % 2].at[pl.ds(0, n)], sem_ei[q % 2])

    def e_compute(q):
        buf = etf2[q % 2]
        eb = eidx2[q % 2]

        @functools.partial(plsc.parallel_loop, 0, e_sz(q) * 8, unroll=2)
        def _(k):
            attr = eb[pl.ds(k * L, L)]
            s = ((k >> 3) << 10) + ((k & 7) << 4)
            for d in range(DE):
                col = plsc.load_gather(ntt_v, [attr + d * EDGE_VOCAB])
                buf[pl.ds(s + (d // 8) * (ECHT * 1024) + (d % 8) * 128,
                          L)] = col

    def e_out_start(q):
        return [pltpu.async_copy(
            etf2[q % 2].at[pl.ds(tr * ECHT * 1024, e_sz(q) * 1024)],
            et_out.at[pl.ds(tr * (ETILES * 1024) + (etb0 + e_loc(q)) * 1024,
                            e_sz(q) * 1024)],
            sem_eo[q % 2]) for tr in range(2)]

    def n_loc(p):
        return p * NAP_CH if p < NAP_NC - 1 else ntail

    def n_sz(p):
        return NAP_CH if p < NAP_NC - 1 else NAP_TAIL

    def n_gather_start(p):
        sz = n_sz(p)
        off = p * NAP_CH
        descs = []
        for j in range(0, sz, 128):
            w = min(128, sz - j)
            descs.append(pltpu.async_copy(
                tbl_hbm.at[nidx_v.at[pl.ds(off + j, w)]],
                rows3[p % 3].at[pl.ds(j, w)], sem_g[p % 3]))
        return descs

    def n_normalize(p):
        buf = rows3[p % 3]

        @functools.partial(plsc.parallel_loop, 0, n_sz(p), unroll=2)
        def _(r):
            for c in range(D // L):
                v = buf[r, pl.ds(c * L, L)]
                buf[r, pl.ds(c * L, L)] = v * scales[c] + shifts[c]

    def n_out_start(p):
        return pltpu.async_copy(
            rows3[p % 3].at[pl.ds(0, n_sz(p))],
            x_out.at[pl.ds(nbase + n_loc(p), n_sz(p))], sem_o[p % 3])

    def edge_step(q):
        ed[q].wait()
        if q >= 2:
            for dsc in od[q - 2]:
                dsc.wait()
        e_compute(q)
        od[q] = e_out_start(q)
        if q + 2 < EAP_NC:
            ed[q + 2] = e_idx_start(q + 2)

    def node_step(p):
        for dsc in gd[p]:
            dsc.wait()
        if p + 2 < NAP_NC:
            if p >= 1:
                nod[p - 1].wait()
            gd[p + 2] = n_gather_start(p + 2)
        n_normalize(p)
        nod[p] = n_out_start(p)

    ni_d = pltpu.async_copy(
        nf_hbm.at[pl.ds(nbase, NAP_FULL)], nidx_v.at[pl.ds(0, NAP_FULL)],
        sem_ni)
    nt_d = pltpu.async_copy(
        nf_hbm.at[pl.ds(nbase + ntail, NAP_TAIL)],
        nidx_v.at[pl.ds(NAP_FULL, NAP_TAIL)], sem_ni)
    ed[0] = e_idx_start(0)
    ed[1] = e_idx_start(1)
    ni_d.wait()
    nt_d.wait()
    gd[0] = n_gather_start(0)
    gd[1] = n_gather_start(1)

    p = 0
    for q in range(EAP_NC):
        edge_step(q)
        if q % 3 == 1 and p < NAP_NC:
            node_step(p)
            p += 1
    while p < NAP_NC:
        node_step(p)
        p += 1

    for q in (EAP_NC - 2, EAP_NC - 1):
        for dsc in od[q]:
            dsc.wait()
    nod[NAP_NC - 2].wait()
    nod[NAP_NC - 1].wait()


_apply_kernel = functools.partial(
    pl.kernel,
    compiler_params=pltpu.CompilerParams(needs_layout_passes=False),
    out_type=(
        jax.ShapeDtypeStruct((N_NODES, D), jnp.float32),
        jax.ShapeDtypeStruct((2 * ETILES * 8 * 128,), jnp.float32),
    ),
    mesh=_mesh,
    scratch_types=[
        pltpu.VMEM((NROW_HI,), jnp.int32),
        pltpu.VMEM((NAP_CH, D), jnp.float32),
        pltpu.VMEM((NAP_CH, D), jnp.float32),
        pltpu.VMEM((NAP_CH, D), jnp.float32),
        pltpu.VMEM((ECHT * 128,), jnp.int32),
        pltpu.VMEM((ECHT * 128,), jnp.int32),
        pltpu.VMEM((2 * ECHT * 8 * 128,), jnp.float32),
        pltpu.VMEM((2 * ECHT * 8 * 128,), jnp.float32),
        pltpu.VMEM((DE * EDGE_VOCAB,), jnp.float32),
        pltpu.VMEM((2 * D,), jnp.float32),
    ] + [pltpu.SemaphoreType.DMA] * 11,
)(_apply_body)


def kernel(node_feat, edge_index, edge_attr, node_table, edge_table,
           node_gamma, node_beta, edge_gamma, edge_beta):
    del edge_index
    nsum, nsq, ecnt = _stats_kernel(node_feat, edge_attr, node_table)

    ns = nsum.sum(axis=0)
    nq = nsq.sum(axis=0)
    nmean = ns / N_NODES
    nvar = jnp.maximum(nq / N_NODES - nmean * nmean, 0.0)
    nscale = node_gamma / jnp.sqrt(nvar + BN_EPS)
    nshift = node_beta - nmean * nscale
    coef = jnp.concatenate([nscale, nshift])

    cnt = ecnt.sum(axis=0)[:EDGE_VOCAB].astype(jnp.float32)
    es = cnt @ edge_table
    eq = cnt @ (edge_table * edge_table)
    emean = es / N_EDGES
    evar = jnp.maximum(eq / N_EDGES - emean * emean, 0.0)
    escale = edge_gamma / jnp.sqrt(evar + BN_EPS)
    nt = edge_table * escale + (edge_beta - emean * escale)
    ntt = nt.T.reshape(-1)

    x, et = _apply_kernel(node_feat, edge_attr, node_table, coef, ntt)
    e = (et.reshape(2, ETILES, 8, 128)
         .transpose(1, 3, 0, 2).reshape(N_EDGES, DE))
    return (x, e)
